# Initial kernel scaffold; baseline (speedup 1.0000x reference)
import jax
import jax.numpy as jnp
from jax.experimental import pallas as pl


def kernel(hidden_states, boundary_prob):
    raise NotImplementedError("write your pallas kernel here")


# speedup vs baseline: 59.3672x; 59.3672x over previous
import functools

import jax
import jax.numpy as jnp
from jax.experimental import pallas as pl
from jax.experimental.pallas import tpu as pltpu

_T = 512


def _scan_body(x_ref, p_ref, o_ref, h_ref, *, block_t):
    s = pl.program_id(1)

    @pl.when(s == 0)
    def _init():
        h_ref[...] = jnp.zeros_like(h_ref)

    x = x_ref[0]
    p_raw = p_ref[0]
    pos = jax.lax.broadcasted_iota(jnp.int32, (block_t, 1), 0) + s * block_t
    mask = (p_raw > 0.5) | (pos == 0)
    p = jnp.clip(p_raw, 1e-4, 1.0 - 1e-4)
    a = jnp.where(mask, 1.0 - p, 1.0)
    w = jnp.where(mask, p, 0.0)

    u = w * x
    d = 1
    while d < block_t:
        a_sh = jnp.concatenate(
            [jnp.ones((d, 1), jnp.float32), a[:-d]], axis=0)
        u_sh = jnp.concatenate(
            [jnp.zeros((d, x.shape[1]), jnp.float32), u[:-d]], axis=0)
        u = a * u_sh + u
        a = a_sh * a
        d *= 2

    out = a * h_ref[...] + u
    o_ref[0] = out
    h_ref[...] = out[block_t - 1:block_t]


def kernel(hidden_states, boundary_prob):
    B, L, D = hidden_states.shape
    T = min(_T, L)
    grid = (B, L // T)
    p3 = boundary_prob[:, :, None]

    return pl.pallas_call(
        functools.partial(_scan_body, block_t=T),
        grid=grid,
        in_specs=[
            pl.BlockSpec((1, T, D), lambda b, s: (b, s, 0)),
            pl.BlockSpec((1, T, 1), lambda b, s: (b, s, 0)),
        ],
        out_specs=pl.BlockSpec((1, T, D), lambda b, s: (b, s, 0)),
        out_shape=jax.ShapeDtypeStruct((B, L, D), jnp.float32),
        scratch_shapes=[pltpu.VMEM((1, D), jnp.float32)],
        compiler_params=pltpu.CompilerParams(
            dimension_semantics=("parallel", "arbitrary"),
        ),
    )(hidden_states, p3)
